# Initial kernel scaffold; baseline (speedup 1.0000x reference)
import jax
import jax.numpy as jnp
from jax.experimental import pallas as pl


def kernel(x):
    raise NotImplementedError("write your pallas kernel here")


# speedup vs baseline: 4.4014x; 4.4014x over previous
import jax
import jax.numpy as jnp
from jax.experimental import pallas as pl
from jax.experimental.pallas import tpu as pltpu

EPS_ = 1e-05
P_ = 4
LANES_ = 64 * P_
R_ = 256


def _shift_in_zeros(v, k):
    return jnp.concatenate([jnp.zeros_like(v[:, :k]), v[:, : LANES_ - k]], axis=1)


def _row_prefix(z):
    q1 = z + _shift_in_zeros(z, 64)
    return q1 + _shift_in_zeros(q1, 128)


def _bcast_last_group(q):
    g = q[:, 192:]
    return jnp.concatenate([g, g, g, g], axis=1)


def _revin_kernel(x_ref, o_ref, c1_ref, c2_ref):
    tb = pl.program_id(1)

    @pl.when(tb == 0)
    def _():
        c1_ref[...] = jnp.zeros_like(c1_ref)
        c2_ref[...] = jnp.zeros_like(c2_ref)

    z = x_ref[0]

    row = jax.lax.broadcasted_iota(jnp.int32, (R_, R_), 0)
    col = jax.lax.broadcasted_iota(jnp.int32, (R_, R_), 1)
    lmat = jnp.where(col < row, 1.0, 0.0).astype(jnp.float32)

    q = _row_prefix(z)
    tot = _bcast_last_group(q)
    e = jax.lax.dot(lmat, tot, precision=jax.lax.Precision.HIGHEST)
    s1 = c1_ref[...] + e + q

    t0 = tb * (P_ * R_)
    rr = jax.lax.broadcasted_iota(jnp.int32, (R_, LANES_), 0)
    ll = jax.lax.broadcasted_iota(jnp.int32, (R_, LANES_), 1)
    n = (t0 + P_ * rr + (ll >> 6) + 1).astype(jnp.float32)
    inv_n = 1.0 / n

    mean = s1 * inv_n
    d = z - mean
    d2 = d * d

    q2 = _row_prefix(d2)
    tot2 = _bcast_last_group(q2)
    e2 = jax.lax.dot(lmat, tot2, precision=jax.lax.Precision.HIGHEST)
    s2 = c2_ref[...] + e2 + q2

    var = s2 * inv_n
    o_ref[0] = d * jax.lax.rsqrt(jnp.maximum(var, EPS_))

    c1_ref[...] = c1_ref[...] + e[R_ - 1 :, :] + tot[R_ - 1 :, :]
    c2_ref[...] = c2_ref[...] + e2[R_ - 1 :, :] + tot2[R_ - 1 :, :]


def kernel(x):
    b, t, c = x.shape
    xr = x.reshape(b, t // P_, LANES_)
    rblocks = xr.shape[1] // R_

    out = pl.pallas_call(
        _revin_kernel,
        grid=(b, rblocks),
        in_specs=[pl.BlockSpec((1, R_, LANES_), lambda i, j: (i, j, 0))],
        out_specs=pl.BlockSpec((1, R_, LANES_), lambda i, j: (i, j, 0)),
        out_shape=jax.ShapeDtypeStruct(xr.shape, x.dtype),
        scratch_shapes=[
            pltpu.VMEM((1, LANES_), jnp.float32),
            pltpu.VMEM((1, LANES_), jnp.float32),
        ],
        compiler_params=pltpu.CompilerParams(
            dimension_semantics=("parallel", "arbitrary"),
        ),
    )(xr)
    return out.reshape(b, t, c)


# speedup vs baseline: 5.2765x; 1.1988x over previous
import jax
import jax.numpy as jnp
from jax.experimental import pallas as pl
from jax.experimental.pallas import tpu as pltpu

EPS_ = 1e-05
P_ = 4
LANES_ = 64 * P_
R_ = 256


def _shift_in_zeros(v, k):
    return jnp.concatenate([jnp.zeros_like(v[:, :k]), v[:, : LANES_ - k]], axis=1)


def _row_prefix(z):
    q1 = z + _shift_in_zeros(z, 64)
    return q1 + _shift_in_zeros(q1, 128)


def _bcast_last_group(q):
    g = q[:, 192:]
    return jnp.concatenate([g, g, g, g], axis=1)


def _tril_matmul(lmat16, v):
    hi = v.astype(jnp.bfloat16)
    lo = (v - hi.astype(jnp.float32)).astype(jnp.bfloat16)
    e_hi = jax.lax.dot(lmat16, hi, preferred_element_type=jnp.float32)
    e_lo = jax.lax.dot(lmat16, lo, preferred_element_type=jnp.float32)
    return e_hi + e_lo


def _revin_kernel(x_ref, o_ref, c1_ref, c2_ref):
    tb = pl.program_id(1)

    @pl.when(tb == 0)
    def _():
        c1_ref[...] = jnp.zeros_like(c1_ref)
        c2_ref[...] = jnp.zeros_like(c2_ref)

    z = x_ref[0]

    row = jax.lax.broadcasted_iota(jnp.int32, (R_, R_), 0)
    col = jax.lax.broadcasted_iota(jnp.int32, (R_, R_), 1)
    lmat16 = jnp.where(col < row, 1.0, 0.0).astype(jnp.bfloat16)

    q = _row_prefix(z)
    tot = _bcast_last_group(q)
    e = _tril_matmul(lmat16, tot)
    s1 = c1_ref[...] + e + q

    t0 = tb * (P_ * R_)
    rr = jax.lax.broadcasted_iota(jnp.int32, (R_, LANES_), 0)
    ll = jax.lax.broadcasted_iota(jnp.int32, (R_, LANES_), 1)
    n = (t0 + P_ * rr + (ll >> 6) + 1).astype(jnp.float32)
    inv_n = 1.0 / n

    mean = s1 * inv_n
    d = z - mean
    d2 = d * d

    q2 = _row_prefix(d2)
    tot2 = _bcast_last_group(q2)
    e2 = _tril_matmul(lmat16, tot2)
    s2 = c2_ref[...] + e2 + q2

    var = s2 * inv_n
    o_ref[0] = d * jax.lax.rsqrt(jnp.maximum(var, EPS_))

    c1_ref[...] = c1_ref[...] + e[R_ - 1 :, :] + tot[R_ - 1 :, :]
    c2_ref[...] = c2_ref[...] + e2[R_ - 1 :, :] + tot2[R_ - 1 :, :]


def kernel(x):
    b, t, c = x.shape
    xr = x.reshape(b, t // P_, LANES_)
    rblocks = xr.shape[1] // R_

    out = pl.pallas_call(
        _revin_kernel,
        grid=(b, rblocks),
        in_specs=[pl.BlockSpec((1, R_, LANES_), lambda i, j: (i, j, 0))],
        out_specs=pl.BlockSpec((1, R_, LANES_), lambda i, j: (i, j, 0)),
        out_shape=jax.ShapeDtypeStruct(xr.shape, x.dtype),
        scratch_shapes=[
            pltpu.VMEM((1, LANES_), jnp.float32),
            pltpu.VMEM((1, LANES_), jnp.float32),
        ],
        compiler_params=pltpu.CompilerParams(
            dimension_semantics=("parallel", "arbitrary"),
        ),
    )(xr)
    return out.reshape(b, t, c)


# speedup vs baseline: 6.1592x; 1.1673x over previous
import jax
import jax.numpy as jnp
from jax.experimental import pallas as pl
from jax.experimental.pallas import tpu as pltpu

EPS_ = 1e-05
P_ = 4
R_ = 256
TB_ = P_ * R_


def _tril_matmul(lmat16, v):
    hi = v.astype(jnp.bfloat16)
    lo = (v - hi.astype(jnp.float32)).astype(jnp.bfloat16)
    e_hi = jax.lax.dot(lmat16, hi, preferred_element_type=jnp.float32)
    e_lo = jax.lax.dot(lmat16, lo, preferred_element_type=jnp.float32)
    return e_hi + e_lo


def _group_prefix(z):
    r = z.shape[0]
    q1 = z + jnp.concatenate(
        [jnp.zeros((r, 1, 128), jnp.float32), z[:, :3, :]], axis=1
    )
    return q1 + jnp.concatenate(
        [jnp.zeros((r, 2, 128), jnp.float32), q1[:, :2, :]], axis=1
    )


def _revin_kernel(x_ref, o_ref, c1_ref, c2_ref):
    tb = pl.program_id(1)

    @pl.when(tb == 0)
    def _():
        c1_ref[...] = jnp.zeros_like(c1_ref)
        c2_ref[...] = jnp.zeros_like(c2_ref)

    z2 = jnp.concatenate([x_ref[0], x_ref[1]], axis=-1)
    z = z2.reshape(R_, P_, 128)

    row = jax.lax.broadcasted_iota(jnp.int32, (R_, R_), 0)
    col = jax.lax.broadcasted_iota(jnp.int32, (R_, R_), 1)
    lmat16 = jnp.where(col < row, 1.0, 0.0).astype(jnp.bfloat16)

    q = _group_prefix(z)
    tot = q[:, P_ - 1, :]
    e = _tril_matmul(lmat16, tot)
    s1 = (c1_ref[...] + e)[:, None, :] + q

    t0 = tb * TB_
    rr = jax.lax.broadcasted_iota(jnp.int32, (R_, P_, 128), 0)
    gg = jax.lax.broadcasted_iota(jnp.int32, (R_, P_, 128), 1)
    n = (t0 + P_ * rr + gg + 1).astype(jnp.float32)
    inv_n = 1.0 / n

    mean = s1 * inv_n
    d = z - mean
    d2 = d * d

    q2 = _group_prefix(d2)
    tot2 = q2[:, P_ - 1, :]
    e2 = _tril_matmul(lmat16, tot2)
    s2 = (c2_ref[...] + e2)[:, None, :] + q2

    var = s2 * inv_n
    out = d * jax.lax.rsqrt(jnp.maximum(var, EPS_))
    o2 = out.reshape(TB_, 128)
    o_ref[0] = o2[:, :64]
    o_ref[1] = o2[:, 64:]

    c1_ref[...] = c1_ref[...] + e[R_ - 1 :, :] + tot[R_ - 1 :, :]
    c2_ref[...] = c2_ref[...] + e2[R_ - 1 :, :] + tot2[R_ - 1 :, :]


def kernel(x):
    b, t, c = x.shape

    out = pl.pallas_call(
        _revin_kernel,
        grid=(b // 2, t // TB_),
        in_specs=[pl.BlockSpec((2, TB_, 64), lambda i, j: (i, j, 0))],
        out_specs=pl.BlockSpec((2, TB_, 64), lambda i, j: (i, j, 0)),
        out_shape=jax.ShapeDtypeStruct(x.shape, x.dtype),
        scratch_shapes=[
            pltpu.VMEM((1, 128), jnp.float32),
            pltpu.VMEM((1, 128), jnp.float32),
        ],
        compiler_params=pltpu.CompilerParams(
            dimension_semantics=("parallel", "arbitrary"),
        ),
    )(x)
    return out


# speedup vs baseline: 18.2662x; 2.9657x over previous
import jax
import jax.numpy as jnp
from jax.experimental import pallas as pl
from jax.experimental.pallas import tpu as pltpu

EPS_ = 1e-05
CH_ = 256
NCH_ = 8
TL_ = CH_ * NCH_


def _hi_lo(v):
    hi = v.astype(jnp.bfloat16)
    lo = (v - hi.astype(jnp.float32)).astype(jnp.bfloat16)
    return hi, lo


def _chunk_prefix(v, umat16):
    hi, lo = _hi_lo(v)
    return jax.lax.dot(hi, umat16, preferred_element_type=jnp.float32) + jax.lax.dot(
        lo, umat16, preferred_element_type=jnp.float32
    )


def _revin_kernel(x_ref, u_ref, o_ref, c1_ref, c2_ref):
    tb = pl.program_id(1)

    @pl.when(tb == 0)
    def _():
        c1_ref[...] = jnp.zeros_like(c1_ref)
        c2_ref[...] = jnp.zeros_like(c2_ref)

    z = x_ref[0]
    umat16 = u_ref[...]
    t0 = tb * TL_

    carry1 = c1_ref[...]
    carry2 = c2_ref[...]
    for c in range(NCH_):
        sl = slice(c * CH_, (c + 1) * CH_)
        zc = z[:, sl]
        s1 = _chunk_prefix(zc, umat16) + carry1
        carry1 = s1[:, CH_ - 1 : CH_]

        nt = jax.lax.broadcasted_iota(jnp.int32, (1, CH_), 1) + (t0 + c * CH_ + 1)
        inv_n = 1.0 / nt.astype(jnp.float32)

        d = zc - s1 * inv_n
        s2 = _chunk_prefix(d * d, umat16) + carry2
        carry2 = s2[:, CH_ - 1 : CH_]

        o_ref[0, :, sl] = d * jax.lax.rsqrt(jnp.maximum(s2 * inv_n, EPS_))

    c1_ref[...] = carry1
    c2_ref[...] = carry2


def kernel(x):
    b, t, c = x.shape
    xt = x.transpose(0, 2, 1)
    umat16 = jnp.triu(jnp.ones((CH_, CH_), jnp.bfloat16))

    out = pl.pallas_call(
        _revin_kernel,
        grid=(b, t // TL_),
        in_specs=[
            pl.BlockSpec((1, c, TL_), lambda i, j: (i, 0, j)),
            pl.BlockSpec((CH_, CH_), lambda i, j: (0, 0)),
        ],
        out_specs=pl.BlockSpec((1, c, TL_), lambda i, j: (i, 0, j)),
        out_shape=jax.ShapeDtypeStruct(xt.shape, x.dtype),
        scratch_shapes=[
            pltpu.VMEM((c, 1), jnp.float32),
            pltpu.VMEM((c, 1), jnp.float32),
        ],
        compiler_params=pltpu.CompilerParams(
            dimension_semantics=("parallel", "arbitrary"),
        ),
    )(xt, umat16)
    return out.transpose(0, 2, 1)


# speedup vs baseline: 20.2345x; 1.1078x over previous
import jax
import jax.numpy as jnp
from jax.experimental import pallas as pl
from jax.experimental.pallas import tpu as pltpu

EPS_ = 1e-05
CH_ = 256
NCH_ = 8
TL_ = CH_ * NCH_


def _revin_kernel(x_ref, u_ref, o_ref, c1_ref, c2_ref):
    tb = pl.program_id(1)

    @pl.when(tb == 0)
    def _():
        c1_ref[...] = jnp.zeros_like(c1_ref)
        c2_ref[...] = jnp.zeros_like(c2_ref)

    z = x_ref[0]
    umat16 = u_ref[...]
    t0 = tb * TL_

    carry1 = c1_ref[...]
    carry2 = c2_ref[...]
    for c in range(NCH_):
        sl = slice(c * CH_, (c + 1) * CH_)
        zc = z[:, sl]
        s1ex = (
            jax.lax.dot(
                zc.astype(jnp.bfloat16), umat16,
                preferred_element_type=jnp.float32,
            )
            + carry1
        )
        carry1 = s1ex[:, CH_ - 1 :] + zc[:, CH_ - 1 :]

        nm1 = jax.lax.broadcasted_iota(jnp.int32, (1, CH_), 1) + (t0 + c * CH_)
        nm1f = nm1.astype(jnp.float32)
        inv_n = 1.0 / (nm1f + 1.0)

        d = (zc * nm1f - s1ex) * inv_n
        d2 = d * d
        s2ex = (
            jax.lax.dot(
                d2.astype(jnp.bfloat16), umat16,
                preferred_element_type=jnp.float32,
            )
            + carry2
        )
        carry2 = s2ex[:, CH_ - 1 :] + d2[:, CH_ - 1 :]

        o_ref[0, :, sl] = d * jax.lax.rsqrt(jnp.maximum((s2ex + d2) * inv_n, EPS_))

    c1_ref[...] = carry1
    c2_ref[...] = carry2


def kernel(x):
    b, t, c = x.shape
    xt = x.transpose(0, 2, 1)
    umat16 = jnp.triu(jnp.ones((CH_, CH_), jnp.bfloat16), k=1)

    out = pl.pallas_call(
        _revin_kernel,
        grid=(b, t // TL_),
        in_specs=[
            pl.BlockSpec((1, c, TL_), lambda i, j: (i, 0, j)),
            pl.BlockSpec((CH_, CH_), lambda i, j: (0, 0)),
        ],
        out_specs=pl.BlockSpec((1, c, TL_), lambda i, j: (i, 0, j)),
        out_shape=jax.ShapeDtypeStruct(xt.shape, x.dtype),
        scratch_shapes=[
            pltpu.VMEM((c, 1), jnp.float32),
            pltpu.VMEM((c, 1), jnp.float32),
        ],
        compiler_params=pltpu.CompilerParams(
            dimension_semantics=("parallel", "arbitrary"),
        ),
    )(xt, umat16)
    return out.transpose(0, 2, 1)


# speedup vs baseline: 32.2848x; 1.5955x over previous
import jax
import jax.numpy as jnp
from jax.experimental import pallas as pl
from jax.experimental.pallas import tpu as pltpu

EPS_ = 1e-05
CH_ = 256
NCH_ = 8
TL_ = CH_ * NCH_
G_ = 2


def _revin_kernel(x_ref, u_ref, o_ref, c1_ref, c2_ref):
    tb = pl.program_id(1)

    @pl.when(tb == 0)
    def _():
        c1_ref[...] = jnp.zeros_like(c1_ref)
        c2_ref[...] = jnp.zeros_like(c2_ref)

    umat16 = u_ref[...]
    t0 = tb * TL_

    carry1 = [c1_ref[:, g : g + 1] for g in range(G_)]
    carry2 = [c2_ref[:, g : g + 1] for g in range(G_)]
    for c in range(NCH_):
        sl = slice(c * CH_, (c + 1) * CH_)
        nm1 = jax.lax.broadcasted_iota(jnp.int32, (1, CH_), 1) + (t0 + c * CH_)
        nm1f = nm1.astype(jnp.float32)
        inv_n = 1.0 / (nm1f + 1.0)
        for g in range(G_):
            zc = x_ref[g][:, sl]
            s1ex = (
                jax.lax.dot(
                    zc.astype(jnp.bfloat16), umat16,
                    preferred_element_type=jnp.float32,
                )
                + carry1[g]
            )
            carry1[g] = s1ex[:, CH_ - 1 :] + zc[:, CH_ - 1 :]

            d = (zc * nm1f - s1ex) * inv_n
            d2 = d * d
            s2ex = (
                jax.lax.dot(
                    d2.astype(jnp.bfloat16), umat16,
                    preferred_element_type=jnp.float32,
                )
                + carry2[g]
            )
            carry2[g] = s2ex[:, CH_ - 1 :] + d2[:, CH_ - 1 :]

            o_ref[g, :, sl] = d * jax.lax.rsqrt(
                jnp.maximum((s2ex + d2) * inv_n, EPS_)
            )

    for g in range(G_):
        c1_ref[:, g : g + 1] = carry1[g]
        c2_ref[:, g : g + 1] = carry2[g]


def kernel(x):
    b, t, c = x.shape
    xt = x.transpose(0, 2, 1)
    umat16 = jnp.triu(jnp.ones((CH_, CH_), jnp.bfloat16), k=1)

    out = pl.pallas_call(
        _revin_kernel,
        grid=(b // G_, t // TL_),
        in_specs=[
            pl.BlockSpec((G_, c, TL_), lambda i, j: (i, 0, j)),
            pl.BlockSpec((CH_, CH_), lambda i, j: (0, 0)),
        ],
        out_specs=pl.BlockSpec((G_, c, TL_), lambda i, j: (i, 0, j)),
        out_shape=jax.ShapeDtypeStruct(xt.shape, x.dtype),
        scratch_shapes=[
            pltpu.VMEM((c, G_), jnp.float32),
            pltpu.VMEM((c, G_), jnp.float32),
        ],
        compiler_params=pltpu.CompilerParams(
            dimension_semantics=("parallel", "arbitrary"),
        ),
    )(xt, umat16)
    return out.transpose(0, 2, 1)


# speedup vs baseline: 44.6119x; 1.3818x over previous
import jax
import jax.numpy as jnp
from jax.experimental import pallas as pl
from jax.experimental.pallas import tpu as pltpu

EPS_ = 1e-05
CH_ = 256
NCH_ = 8
TL_ = CH_ * NCH_
G_ = 4


def _revin_kernel(x_ref, u_ref, o_ref, c1_ref, c2_ref):
    tb = pl.program_id(1)

    @pl.when(tb == 0)
    def _():
        c1_ref[...] = jnp.zeros_like(c1_ref)
        c2_ref[...] = jnp.zeros_like(c2_ref)

    umat16 = u_ref[...]
    t0 = tb * TL_

    carry1 = [c1_ref[:, g : g + 1] for g in range(G_)]
    carry2 = [c2_ref[:, g : g + 1] for g in range(G_)]
    for c in range(NCH_):
        sl = slice(c * CH_, (c + 1) * CH_)
        nm1 = jax.lax.broadcasted_iota(jnp.int32, (1, CH_), 1) + (t0 + c * CH_)
        nm1f = nm1.astype(jnp.float32)
        inv_n = 1.0 / (nm1f + 1.0)
        for g in range(G_):
            zc = x_ref[g][:, sl]
            s1ex = (
                jax.lax.dot(
                    zc.astype(jnp.bfloat16), umat16,
                    preferred_element_type=jnp.float32,
                )
                + carry1[g]
            )
            carry1[g] = s1ex[:, CH_ - 1 :] + zc[:, CH_ - 1 :]

            d = (zc * nm1f - s1ex) * inv_n
            d2 = d * d
            s2ex = (
                jax.lax.dot(
                    d2.astype(jnp.bfloat16), umat16,
                    preferred_element_type=jnp.float32,
                )
                + carry2[g]
            )
            carry2[g] = s2ex[:, CH_ - 1 :] + d2[:, CH_ - 1 :]

            o_ref[g, :, sl] = d * jax.lax.rsqrt(
                jnp.maximum((s2ex + d2) * inv_n, EPS_)
            )

    for g in range(G_):
        c1_ref[:, g : g + 1] = carry1[g]
        c2_ref[:, g : g + 1] = carry2[g]


def kernel(x):
    b, t, c = x.shape
    xt = x.transpose(0, 2, 1)
    umat16 = jnp.triu(jnp.ones((CH_, CH_), jnp.bfloat16), k=1)

    out = pl.pallas_call(
        _revin_kernel,
        grid=(b // G_, t // TL_),
        in_specs=[
            pl.BlockSpec((G_, c, TL_), lambda i, j: (i, 0, j)),
            pl.BlockSpec((CH_, CH_), lambda i, j: (0, 0)),
        ],
        out_specs=pl.BlockSpec((G_, c, TL_), lambda i, j: (i, 0, j)),
        out_shape=jax.ShapeDtypeStruct(xt.shape, x.dtype),
        scratch_shapes=[
            pltpu.VMEM((c, G_), jnp.float32),
            pltpu.VMEM((c, G_), jnp.float32),
        ],
        compiler_params=pltpu.CompilerParams(
            dimension_semantics=("parallel", "arbitrary"),
        ),
    )(xt, umat16)
    return out.transpose(0, 2, 1)


# speedup vs baseline: 55.5159x; 1.2444x over previous
import jax
import jax.numpy as jnp
from jax.experimental import pallas as pl
from jax.experimental.pallas import tpu as pltpu

EPS_ = 1e-05
CH_ = 256
NCH_ = 8
TL_ = CH_ * NCH_
G_ = 8


def _revin_kernel(x_ref, u_ref, o_ref, c1_ref, c2_ref):
    tb = pl.program_id(1)

    @pl.when(tb == 0)
    def _():
        c1_ref[...] = jnp.zeros_like(c1_ref)
        c2_ref[...] = jnp.zeros_like(c2_ref)

    umat16 = u_ref[...]
    t0 = tb * TL_

    carry1 = [c1_ref[:, g : g + 1] for g in range(G_)]
    carry2 = [c2_ref[:, g : g + 1] for g in range(G_)]
    for c in range(NCH_):
        sl = slice(c * CH_, (c + 1) * CH_)
        nm1 = jax.lax.broadcasted_iota(jnp.int32, (1, CH_), 1) + (t0 + c * CH_)
        nm1f = nm1.astype(jnp.float32)
        inv_n = 1.0 / (nm1f + 1.0)
        for g in range(G_):
            zc = x_ref[g][:, sl]
            s1ex = (
                jax.lax.dot(
                    zc.astype(jnp.bfloat16), umat16,
                    preferred_element_type=jnp.float32,
                )
                + carry1[g]
            )
            carry1[g] = s1ex[:, CH_ - 1 :] + zc[:, CH_ - 1 :]

            d = (zc * nm1f - s1ex) * inv_n
            d2 = d * d
            s2ex = (
                jax.lax.dot(
                    d2.astype(jnp.bfloat16), umat16,
                    preferred_element_type=jnp.float32,
                )
                + carry2[g]
            )
            carry2[g] = s2ex[:, CH_ - 1 :] + d2[:, CH_ - 1 :]

            o_ref[g, :, sl] = d * jax.lax.rsqrt(
                jnp.maximum((s2ex + d2) * inv_n, EPS_)
            )

    for g in range(G_):
        c1_ref[:, g : g + 1] = carry1[g]
        c2_ref[:, g : g + 1] = carry2[g]


def kernel(x):
    b, t, c = x.shape
    xt = x.transpose(0, 2, 1)
    umat16 = jnp.triu(jnp.ones((CH_, CH_), jnp.bfloat16), k=1)

    out = pl.pallas_call(
        _revin_kernel,
        grid=(b // G_, t // TL_),
        in_specs=[
            pl.BlockSpec((G_, c, TL_), lambda i, j: (i, 0, j)),
            pl.BlockSpec((CH_, CH_), lambda i, j: (0, 0)),
        ],
        out_specs=pl.BlockSpec((G_, c, TL_), lambda i, j: (i, 0, j)),
        out_shape=jax.ShapeDtypeStruct(xt.shape, x.dtype),
        scratch_shapes=[
            pltpu.VMEM((c, G_), jnp.float32),
            pltpu.VMEM((c, G_), jnp.float32),
        ],
        compiler_params=pltpu.CompilerParams(
            dimension_semantics=("parallel", "arbitrary"),
        ),
    )(xt, umat16)
    return out.transpose(0, 2, 1)


# speedup vs baseline: 60.9079x; 1.0971x over previous
import jax
import jax.numpy as jnp
from jax.experimental import pallas as pl
from jax.experimental.pallas import tpu as pltpu

EPS_ = 1e-05
CH_ = 256
NCH_ = 8
TL_ = CH_ * NCH_
G_ = 16


def _revin_kernel(x_ref, u_ref, o_ref, c1_ref, c2_ref):
    tb = pl.program_id(1)

    @pl.when(tb == 0)
    def _():
        c1_ref[...] = jnp.zeros_like(c1_ref)
        c2_ref[...] = jnp.zeros_like(c2_ref)

    umat16 = u_ref[...]
    t0 = tb * TL_

    carry1 = [c1_ref[:, g : g + 1] for g in range(G_)]
    carry2 = [c2_ref[:, g : g + 1] for g in range(G_)]
    for c in range(NCH_):
        sl = slice(c * CH_, (c + 1) * CH_)
        nm1 = jax.lax.broadcasted_iota(jnp.int32, (1, CH_), 1) + (t0 + c * CH_)
        nm1f = nm1.astype(jnp.float32)
        inv_n = 1.0 / (nm1f + 1.0)
        for g in range(G_):
            zc = x_ref[g][:, sl]
            s1ex = (
                jax.lax.dot(
                    zc.astype(jnp.bfloat16), umat16,
                    preferred_element_type=jnp.float32,
                )
                + carry1[g]
            )
            carry1[g] = s1ex[:, CH_ - 1 :] + zc[:, CH_ - 1 :]

            d = (zc * nm1f - s1ex) * inv_n
            d2 = d * d
            s2ex = (
                jax.lax.dot(
                    d2.astype(jnp.bfloat16), umat16,
                    preferred_element_type=jnp.float32,
                )
                + carry2[g]
            )
            carry2[g] = s2ex[:, CH_ - 1 :] + d2[:, CH_ - 1 :]

            o_ref[g, :, sl] = d * jax.lax.rsqrt(
                jnp.maximum((s2ex + d2) * inv_n, EPS_)
            )

    for g in range(G_):
        c1_ref[:, g : g + 1] = carry1[g]
        c2_ref[:, g : g + 1] = carry2[g]


def kernel(x):
    b, t, c = x.shape
    xt = x.transpose(0, 2, 1)
    umat16 = jnp.triu(jnp.ones((CH_, CH_), jnp.bfloat16), k=1)

    out = pl.pallas_call(
        _revin_kernel,
        grid=(b // G_, t // TL_),
        in_specs=[
            pl.BlockSpec((G_, c, TL_), lambda i, j: (i, 0, j)),
            pl.BlockSpec((CH_, CH_), lambda i, j: (0, 0)),
        ],
        out_specs=pl.BlockSpec((G_, c, TL_), lambda i, j: (i, 0, j)),
        out_shape=jax.ShapeDtypeStruct(xt.shape, x.dtype),
        scratch_shapes=[
            pltpu.VMEM((c, G_), jnp.float32),
            pltpu.VMEM((c, G_), jnp.float32),
        ],
        compiler_params=pltpu.CompilerParams(
            dimension_semantics=("parallel", "arbitrary"),
        ),
    )(xt, umat16)
    return out.transpose(0, 2, 1)
